# Initial kernel scaffold; baseline (speedup 1.0000x reference)
import jax
import jax.numpy as jnp
from jax.experimental import pallas as pl


def kernel(hidden_states, Wq, bq, Wk, bk, Wv, bv, Wp, log_bandwidth):
    raise NotImplementedError("write your pallas kernel here")


# speedup vs baseline: 9.5237x; 9.5237x over previous
import functools

import jax
import jax.numpy as jnp
from jax.experimental import pallas as pl


def _qkv_mm_kernel(a_ref, w_ref, b_ref, o_ref):
    acc = jnp.dot(a_ref[...], w_ref[...], preferred_element_type=jnp.float32)
    o_ref[...] = acc + b_ref[...]


def _z_mm_kernel(a_ref, w_ref, o_ref):
    o_ref[...] = jnp.dot(a_ref[...], w_ref[...], preferred_element_type=jnp.float32)


def _attn_kernel(zq_ref, zk_ref, v_ref, o_ref, *, inv_t):
    zq = zq_ref[0]
    zk = zk_ref[0]
    w = jnp.maximum(1.0 - jnp.abs(zq - zk), 0.0)
    o_ref[0] = jnp.dot(w, v_ref[0], preferred_element_type=jnp.float32) * inv_t


def kernel(hidden_states, Wq, bq, Wk, bk, Wv, bv, Wp, log_bandwidth):
    f32 = jnp.float32
    Bs, T, Hid = hidden_states.shape
    H = Wp.shape[0]
    D = Hid // H
    BH = Bs * H
    M = Bs * T
    N = 3 * Hid

    hs2 = hidden_states.reshape(M, Hid).astype(f32)
    W_all = jnp.concatenate([Wq, Wk, Wv], axis=0).T.astype(f32)
    b_all = jnp.concatenate([bq, bk, bv])[None, :].astype(f32)

    TM = min(512, M)
    TN = min(512, N)
    qkv = pl.pallas_call(
        _qkv_mm_kernel,
        grid=(M // TM, N // TN),
        in_specs=[
            pl.BlockSpec((TM, Hid), lambda i, j: (i, 0)),
            pl.BlockSpec((Hid, TN), lambda i, j: (0, j)),
            pl.BlockSpec((1, TN), lambda i, j: (0, j)),
        ],
        out_specs=pl.BlockSpec((TM, TN), lambda i, j: (i, j)),
        out_shape=jax.ShapeDtypeStruct((M, N), f32),
    )(hs2, W_all, b_all)

    q, k, v = qkv[:, :Hid], qkv[:, Hid:2 * Hid], qkv[:, 2 * Hid:]

    def scr(x):
        return x.reshape(Bs, T, H, D).transpose(0, 2, 1, 3).reshape(Bs, T, H * D)

    X = jnp.concatenate([scr(q), scr(k)], axis=0).reshape(2 * M, Hid)

    bw = jax.nn.softplus(log_bandwidth.astype(f32)) + 1e-4
    Wp_s = (Wp.astype(f32) / bw[:, None]).T
    NP = 128
    Wp_pad = jnp.zeros((Hid, NP), f32).at[:, :H].set(Wp_s)

    TMZ = min(512, 2 * M)
    z = pl.pallas_call(
        _z_mm_kernel,
        grid=(2 * M // TMZ,),
        in_specs=[
            pl.BlockSpec((TMZ, Hid), lambda i: (i, 0)),
            pl.BlockSpec((Hid, NP), lambda i: (0, 0)),
        ],
        out_specs=pl.BlockSpec((TMZ, NP), lambda i: (i, 0)),
        out_shape=jax.ShapeDtypeStruct((2 * M, NP), f32),
    )(X, Wp_pad)

    z = z[:, :H]
    zq = z[:M].reshape(Bs, T, H).transpose(0, 2, 1).reshape(BH, T, 1)
    zk = z[M:].reshape(Bs, T, H).transpose(0, 2, 1).reshape(BH, 1, T)
    vh = v.reshape(Bs, T, H, D).transpose(0, 2, 1, 3).reshape(BH, T, D)

    TQ = min(512, T)
    ctx = pl.pallas_call(
        functools.partial(_attn_kernel, inv_t=1.0 / T),
        grid=(BH, T // TQ),
        in_specs=[
            pl.BlockSpec((1, TQ, 1), lambda b, i: (b, i, 0)),
            pl.BlockSpec((1, 1, T), lambda b, i: (b, 0, 0)),
            pl.BlockSpec((1, T, D), lambda b, i: (b, 0, 0)),
        ],
        out_specs=pl.BlockSpec((1, TQ, D), lambda b, i: (b, i, 0)),
        out_shape=jax.ShapeDtypeStruct((BH, T, D), f32),
    )(zq, zk, vh)

    return ctx.reshape(Bs, H, T, D).transpose(0, 2, 1, 3).reshape(Bs, T, H * D)


# speedup vs baseline: 13.8526x; 1.4545x over previous
import functools

import jax
import jax.numpy as jnp
from jax.experimental import pallas as pl


def _qkv_kernel(hs_ref, wq_ref, wk_ref, wv_ref, bq_ref, bk_ref, bv_ref,
                q_ref, k_ref, v_ref):
    a = hs_ref[...]
    f32 = jnp.float32
    q_ref[0, 0] = jnp.dot(a, wq_ref[...], preferred_element_type=f32) + bq_ref[...]
    k_ref[0, 0] = jnp.dot(a, wk_ref[...], preferred_element_type=f32) + bk_ref[...]
    v_ref[0, 0] = jnp.dot(a, wv_ref[...], preferred_element_type=f32) + bv_ref[...]


def _z_kernel(xq_ref, xk_ref, wp_ref, zq_ref, zk_ref):
    wp = wp_ref[...]
    zq_ref[...] = jnp.dot(xq_ref[...], wp, preferred_element_type=jnp.float32)
    zk_ref[...] = jnp.dot(xk_ref[...], wp, preferred_element_type=jnp.float32)


def _attn_kernel(zq_ref, zk_ref, v_ref, o_ref, *, inv_t):
    zq = zq_ref[0]
    zk = zk_ref[0]
    w = jnp.maximum(1.0 - jnp.abs(zq - zk), 0.0)
    o_ref[0] = jnp.dot(w, v_ref[0], preferred_element_type=jnp.float32) * inv_t


def kernel(hidden_states, Wq, bq, Wk, bk, Wv, bv, Wp, log_bandwidth):
    f32 = jnp.float32
    Bs, T, Hid = hidden_states.shape
    H = Wp.shape[0]
    D = Hid // H
    HB = H * Bs
    M = Bs * T
    R = T // H

    hs2 = hidden_states.reshape(M, Hid).astype(f32)
    WqT = Wq.T.astype(f32)
    WkT = Wk.T.astype(f32)
    WvT = Wv.T.astype(f32)
    bq2 = bq[None, :].astype(f32)
    bk2 = bk[None, :].astype(f32)
    bv2 = bv[None, :].astype(f32)

    TM = min(1024, T)
    n_t = T // TM
    w_spec = pl.BlockSpec((Hid, D), lambda i, h: (0, h))
    b_spec = pl.BlockSpec((1, D), lambda i, h: (0, h))
    o_spec = pl.BlockSpec((1, 1, TM, D),
                          lambda i, h: (h, i // n_t, i % n_t, 0))
    qh, kh, vh = pl.pallas_call(
        _qkv_kernel,
        grid=(M // TM, H),
        in_specs=[
            pl.BlockSpec((TM, Hid), lambda i, h: (i, 0)),
            w_spec, w_spec, w_spec, b_spec, b_spec, b_spec,
        ],
        out_specs=[o_spec, o_spec, o_spec],
        out_shape=[jax.ShapeDtypeStruct((H, Bs, T, D), f32)] * 3,
    )(hs2, WqT, WkT, WvT, bq2, bk2, bv2)

    xq = qh.reshape(HB * R, Hid)
    xk = kh.reshape(HB * R, Hid)

    bw = jax.nn.softplus(log_bandwidth.astype(f32)) + 1e-4
    Wp_s = (Wp.astype(f32) / bw[:, None]).T
    NP = 128
    Wp_pad = jnp.zeros((Hid, NP), f32).at[:, :H].set(Wp_s)

    TMZ = min(512, HB * R)
    zq_r, zk_r = pl.pallas_call(
        _z_kernel,
        grid=(HB * R // TMZ,),
        in_specs=[
            pl.BlockSpec((TMZ, Hid), lambda i: (i, 0)),
            pl.BlockSpec((TMZ, Hid), lambda i: (i, 0)),
            pl.BlockSpec((Hid, NP), lambda i: (0, 0)),
        ],
        out_specs=[
            pl.BlockSpec((TMZ, NP), lambda i: (i, 0)),
            pl.BlockSpec((TMZ, NP), lambda i: (i, 0)),
        ],
        out_shape=[jax.ShapeDtypeStruct((HB * R, NP), f32)] * 2,
    )(xq, xk, Wp_pad)

    def to_hb(z_r):
        z4 = z_r.reshape(H, Bs, R, NP)[..., :H]
        return z4.transpose(3, 1, 0, 2).reshape(HB, T)

    zq = to_hb(zq_r).reshape(HB, T, 1)
    zk = to_hb(zk_r).reshape(HB, 1, T)
    v3 = vh.reshape(HB, T, D)

    TQ = min(512, T)
    out = pl.pallas_call(
        functools.partial(_attn_kernel, inv_t=1.0 / T),
        grid=(HB, T // TQ),
        in_specs=[
            pl.BlockSpec((1, TQ, 1), lambda g, i: (g, i, 0)),
            pl.BlockSpec((1, 1, T), lambda g, i: (g, 0, 0)),
            pl.BlockSpec((1, T, D), lambda g, i: (g, 0, 0)),
        ],
        out_specs=pl.BlockSpec((1, TQ, D),
                               lambda g, i: (g % Bs, i, g // Bs)),
        out_shape=jax.ShapeDtypeStruct((Bs, T, Hid), f32),
    )(zq, zk, v3)

    return out


# speedup vs baseline: 14.6670x; 1.0588x over previous
import functools

import jax
import jax.numpy as jnp
from jax.experimental import pallas as pl


def _qkv_kernel(hs_ref, wq_ref, wk_ref, wv_ref, bq_ref, bk_ref, bv_ref,
                q_ref, k_ref, v_ref):
    a = hs_ref[...]
    f32 = jnp.float32
    bf16 = jnp.bfloat16
    q = jnp.dot(a, wq_ref[...], preferred_element_type=f32) + bq_ref[...]
    k = jnp.dot(a, wk_ref[...], preferred_element_type=f32) + bk_ref[...]
    v = jnp.dot(a, wv_ref[...], preferred_element_type=f32) + bv_ref[...]
    q_ref[0, 0] = q.astype(bf16)
    k_ref[0, 0] = k.astype(bf16)
    v_ref[0, 0] = v.astype(bf16)


def _z_kernel(xq_ref, xk_ref, wp_ref, zq_ref, zk_ref):
    wp = wp_ref[...]
    zq_ref[...] = jnp.dot(xq_ref[...], wp, preferred_element_type=jnp.float32)
    zk_ref[...] = jnp.dot(xk_ref[...], wp, preferred_element_type=jnp.float32)


def _attn_kernel(zq_ref, zk_ref, v_ref, o_ref, *, inv_t):
    zq = zq_ref[0]
    zk = zk_ref[0]
    w = jnp.maximum(1.0 - jnp.abs(zq - zk), 0.0).astype(jnp.bfloat16)
    o_ref[0] = jnp.dot(w, v_ref[0], preferred_element_type=jnp.float32) * inv_t


def kernel(hidden_states, Wq, bq, Wk, bk, Wv, bv, Wp, log_bandwidth):
    f32 = jnp.float32
    Bs, T, Hid = hidden_states.shape
    H = Wp.shape[0]
    D = Hid // H
    HB = H * Bs
    M = Bs * T
    R = T // H

    bf16 = jnp.bfloat16
    hs2 = hidden_states.reshape(M, Hid).astype(bf16)
    WqT = Wq.T.astype(bf16)
    WkT = Wk.T.astype(bf16)
    WvT = Wv.T.astype(bf16)
    bq2 = bq[None, :].astype(f32)
    bk2 = bk[None, :].astype(f32)
    bv2 = bv[None, :].astype(f32)

    TM = min(1024, T)
    n_t = T // TM
    w_spec = pl.BlockSpec((Hid, D), lambda i, h: (0, h))
    b_spec = pl.BlockSpec((1, D), lambda i, h: (0, h))
    o_spec = pl.BlockSpec((1, 1, TM, D),
                          lambda i, h: (h, i // n_t, i % n_t, 0))
    qh, kh, vh = pl.pallas_call(
        _qkv_kernel,
        grid=(M // TM, H),
        in_specs=[
            pl.BlockSpec((TM, Hid), lambda i, h: (i, 0)),
            w_spec, w_spec, w_spec, b_spec, b_spec, b_spec,
        ],
        out_specs=[o_spec, o_spec, o_spec],
        out_shape=[jax.ShapeDtypeStruct((H, Bs, T, D), bf16)] * 3,
    )(hs2, WqT, WkT, WvT, bq2, bk2, bv2)

    xq = qh.reshape(HB * R, Hid)
    xk = kh.reshape(HB * R, Hid)

    bw = jax.nn.softplus(log_bandwidth.astype(f32)) + 1e-4
    Wp_s = (Wp.astype(f32) / bw[:, None]).T
    NP = 128
    Wp_pad = jnp.zeros((Hid, NP), f32).at[:, :H].set(Wp_s).astype(bf16)

    TMZ = min(512, HB * R)
    zq_r, zk_r = pl.pallas_call(
        _z_kernel,
        grid=(HB * R // TMZ,),
        in_specs=[
            pl.BlockSpec((TMZ, Hid), lambda i: (i, 0)),
            pl.BlockSpec((TMZ, Hid), lambda i: (i, 0)),
            pl.BlockSpec((Hid, NP), lambda i: (0, 0)),
        ],
        out_specs=[
            pl.BlockSpec((TMZ, NP), lambda i: (i, 0)),
            pl.BlockSpec((TMZ, NP), lambda i: (i, 0)),
        ],
        out_shape=[jax.ShapeDtypeStruct((HB * R, NP), f32)] * 2,
    )(xq, xk, Wp_pad)

    def to_hb(z_r):
        z4 = z_r.reshape(H, Bs, R, NP)[..., :H]
        return z4.transpose(3, 1, 0, 2).reshape(HB, T)

    zq = to_hb(zq_r).reshape(HB, T, 1)
    zk = to_hb(zk_r).reshape(HB, 1, T)
    v3 = vh.reshape(HB, T, D)

    TQ = min(512, T)
    out = pl.pallas_call(
        functools.partial(_attn_kernel, inv_t=1.0 / T),
        grid=(HB, T // TQ),
        in_specs=[
            pl.BlockSpec((1, TQ, 1), lambda g, i: (g, i, 0)),
            pl.BlockSpec((1, 1, T), lambda g, i: (g, 0, 0)),
            pl.BlockSpec((1, T, D), lambda g, i: (g, 0, 0)),
        ],
        out_specs=pl.BlockSpec((1, TQ, D),
                               lambda g, i: (g % Bs, i, g // Bs)),
        out_shape=jax.ShapeDtypeStruct((Bs, T, Hid), f32),
    )(zq, zk, v3)

    return out


# speedup vs baseline: 20.4888x; 1.3969x over previous
import functools

import jax
import jax.numpy as jnp
from jax.experimental import pallas as pl


def _qkv_kernel(hs_ref, wq_ref, wk_ref, wv_ref, bq_ref, bk_ref, bv_ref,
                q_ref, k_ref, v_ref, *, hp, d):
    a = hs_ref[...]
    f32 = jnp.float32
    bf16 = jnp.bfloat16
    q = (jnp.dot(a, wq_ref[...], preferred_element_type=f32) + bq_ref[...]).astype(bf16)
    k = (jnp.dot(a, wk_ref[...], preferred_element_type=f32) + bk_ref[...]).astype(bf16)
    v = (jnp.dot(a, wv_ref[...], preferred_element_type=f32) + bv_ref[...]).astype(bf16)
    for hh in range(hp):
        sl = slice(hh * d, (hh + 1) * d)
        q_ref[hh, 0] = q[:, sl]
        k_ref[hh, 0] = k[:, sl]
        v_ref[hh, 0] = v[:, sl]


def _z_kernel(xq_ref, xk_ref, wp_ref, zq_ref, zk_ref):
    wp = wp_ref[...]
    zq_ref[...] = jnp.dot(xq_ref[...], wp, preferred_element_type=jnp.float32)
    zk_ref[...] = jnp.dot(xk_ref[...], wp, preferred_element_type=jnp.float32)


def _attn_kernel(zq_ref, zk_ref, v_ref, o_ref, *, inv_t):
    zq = zq_ref[0]
    zk = zk_ref[0]
    w = jnp.maximum(1.0 - jnp.abs(zq - zk), 0.0).astype(jnp.bfloat16)
    o_ref[0] = jnp.dot(w, v_ref[0], preferred_element_type=jnp.float32) * inv_t


def kernel(hidden_states, Wq, bq, Wk, bk, Wv, bv, Wp, log_bandwidth):
    f32 = jnp.float32
    Bs, T, Hid = hidden_states.shape
    H = Wp.shape[0]
    D = Hid // H
    HB = H * Bs
    M = Bs * T
    R = T // H

    bf16 = jnp.bfloat16
    hs2 = hidden_states.reshape(M, Hid).astype(bf16)
    WqT = Wq.T.astype(bf16)
    WkT = Wk.T.astype(bf16)
    WvT = Wv.T.astype(bf16)
    bq2 = bq[None, :].astype(f32)
    bk2 = bk[None, :].astype(f32)
    bv2 = bv[None, :].astype(f32)

    TM = min(1024, T)
    n_t = T // TM
    HP = min(8, H)
    w_spec = pl.BlockSpec((Hid, HP * D), lambda h, i: (0, h))
    b_spec = pl.BlockSpec((1, HP * D), lambda h, i: (0, h))
    o_spec = pl.BlockSpec((HP, 1, TM, D),
                          lambda h, i: (h, i // n_t, i % n_t, 0))
    qh, kh, vh = pl.pallas_call(
        functools.partial(_qkv_kernel, hp=HP, d=D),
        grid=(H // HP, M // TM),
        in_specs=[
            pl.BlockSpec((TM, Hid), lambda h, i: (i, 0)),
            w_spec, w_spec, w_spec, b_spec, b_spec, b_spec,
        ],
        out_specs=[o_spec, o_spec, o_spec],
        out_shape=[jax.ShapeDtypeStruct((H, Bs, T, D), bf16)] * 3,
    )(hs2, WqT, WkT, WvT, bq2, bk2, bv2)

    xq = qh.reshape(HB * R, Hid)
    xk = kh.reshape(HB * R, Hid)

    bw = jax.nn.softplus(log_bandwidth.astype(f32)) + 1e-4
    Wp_s = (Wp.astype(f32) / bw[:, None]).T
    NP = 128
    Wp_pad = jnp.zeros((Hid, NP), f32).at[:, :H].set(Wp_s).astype(bf16)

    TMZ = min(512, HB * R)
    zq_r, zk_r = pl.pallas_call(
        _z_kernel,
        grid=(HB * R // TMZ,),
        in_specs=[
            pl.BlockSpec((TMZ, Hid), lambda i: (i, 0)),
            pl.BlockSpec((TMZ, Hid), lambda i: (i, 0)),
            pl.BlockSpec((Hid, NP), lambda i: (0, 0)),
        ],
        out_specs=[
            pl.BlockSpec((TMZ, NP), lambda i: (i, 0)),
            pl.BlockSpec((TMZ, NP), lambda i: (i, 0)),
        ],
        out_shape=[jax.ShapeDtypeStruct((HB * R, NP), f32)] * 2,
    )(xq, xk, Wp_pad)

    def to_hb(z_r):
        z4 = z_r.reshape(H, Bs, R, NP)[..., :H]
        return z4.transpose(3, 1, 0, 2).reshape(HB, T)

    zq = to_hb(zq_r).reshape(HB, T, 1)
    zk = to_hb(zk_r).reshape(HB, 1, T)
    v3 = vh.reshape(HB, T, D)

    TQ = min(1024, T)
    out = pl.pallas_call(
        functools.partial(_attn_kernel, inv_t=1.0 / T),
        grid=(HB, T // TQ),
        in_specs=[
            pl.BlockSpec((1, TQ, 1), lambda g, i: (g, i, 0)),
            pl.BlockSpec((1, 1, T), lambda g, i: (g, 0, 0)),
            pl.BlockSpec((1, T, D), lambda g, i: (g, 0, 0)),
        ],
        out_specs=pl.BlockSpec((1, TQ, D),
                               lambda g, i: (g % Bs, i, g // Bs)),
        out_shape=jax.ShapeDtypeStruct((Bs, T, Hid), f32),
    )(zq, zk, v3)

    return out
